# Initial kernel scaffold; baseline (speedup 1.0000x reference)
import jax
import jax.numpy as jnp
from jax.experimental import pallas as pl


def kernel(x, edge_index, batch, W1, W2, L1_w, L1_b, L2_w, L2_b):
    raise NotImplementedError("write your pallas kernel here")


# speedup vs baseline: 10.0428x; 10.0428x over previous
import functools

import jax
import jax.numpy as jnp
from jax import lax
from jax.experimental import pallas as pl
from jax.experimental.pallas import tpu as pltpu
from jax.experimental.pallas import tpu_sc as plsc

NC = 2
NS = 16
NW = NC * NS
K = 128

_MESH = plsc.VectorSubcoreMesh(core_axis_name="c", subcore_axis_name="s")


def _deg_body(np_, ch, dst_hbm, ones_hbm, zeros_hbm, out_hbm,
              didx, ones_v, acc_sh, sem):
    c = lax.axis_index("c")
    s = lax.axis_index("s")
    w = c * NS + s
    rows = np_ // NS

    pltpu.sync_copy(ones_hbm, ones_v)
    pltpu.sync_copy(zeros_hbm, acc_sh.at[pl.ds(s * rows, rows)])
    plsc.subcore_barrier()

    def chunk(j, carry):
        pltpu.sync_copy(dst_hbm.at[w, j], didx)
        pltpu.sync_copy(ones_v, acc_sh.at[didx], add=True)
        return carry

    lax.fori_loop(0, ch, chunk, 0)
    plsc.subcore_barrier()
    pltpu.sync_copy(acc_sh.at[pl.ds(s * rows, rows)],
                    out_hbm.at[c, pl.ds(s * rows, rows)])


def _make_deg_kernel(np_, ch):
    return functools.partial(
        pl.kernel,
        out_type=jax.ShapeDtypeStruct((NC, np_, 16), jnp.float32),
        mesh=_MESH,
        scratch_types=[
            pltpu.VMEM((K,), jnp.int32),
            pltpu.VMEM((K, 16), jnp.float32),
            pltpu.VMEM_SHARED((np_, 16), jnp.float32),
            pltpu.SemaphoreType.DMA,
        ],
        compiler_params=pltpu.CompilerParams(use_tc_tiling_on_sc=False),
    )(functools.partial(_deg_body, np_, ch))


def _agg_body(np_, ch, dw, g_hbm, src_hbm, dst_hbm, zeros_hbm, out_hbm,
              sidx0, sidx1, didx0, msg0, msg1, acc_sh, gsem0, gsem1):
    c = lax.axis_index("c")
    s = lax.axis_index("s")
    w = c * NS + s
    rows = np_ // NS

    for r in range(rows // K):
        pltpu.sync_copy(zeros_hbm,
                        acc_sh.at[pl.ds(s * rows + r * K, K)])
    plsc.subcore_barrier()

    pltpu.sync_copy(src_hbm.at[w, 0], sidx0)
    pltpu.async_copy(g_hbm.at[sidx0], msg0, gsem0)

    def chunk(j, carry):
        def do(sidx_n, msg_n, gsem_n, sidx_c, msg_c, gsem_c):
            @pl.when(j + 1 < ch)
            def _():
                pltpu.sync_copy(src_hbm.at[w, j + 1], sidx_n)
                pltpu.async_copy(g_hbm.at[sidx_n], msg_n, gsem_n)
            pltpu.make_async_copy(g_hbm.at[sidx_c], msg_c, gsem_c).wait()
            pltpu.sync_copy(dst_hbm.at[w, j], didx0)
            pltpu.sync_copy(msg_c, acc_sh.at[didx0], add=True)

        @pl.when(j % 2 == 0)
        def _():
            do(sidx1, msg1, gsem1, sidx0, msg0, gsem0)

        @pl.when(j % 2 == 1)
        def _():
            do(sidx0, msg0, gsem0, sidx1, msg1, gsem1)

        return carry

    lax.fori_loop(0, ch, chunk, 0)
    plsc.subcore_barrier()
    pltpu.sync_copy(acc_sh.at[pl.ds(s * rows, rows)],
                    out_hbm.at[c, pl.ds(s * rows, rows)])


def _make_agg_kernel(np_, ch, dw):
    return functools.partial(
        pl.kernel,
        out_type=jax.ShapeDtypeStruct((NC, np_, dw), jnp.float32),
        mesh=_MESH,
        scratch_types=[
            pltpu.VMEM((K,), jnp.int32),
            pltpu.VMEM((K,), jnp.int32),
            pltpu.VMEM((K,), jnp.int32),
            pltpu.VMEM((K, dw), jnp.float32),
            pltpu.VMEM((K, dw), jnp.float32),
            pltpu.VMEM_SHARED((np_, dw), jnp.float32),
            pltpu.SemaphoreType.DMA,
            pltpu.SemaphoreType.DMA,
        ],
        compiler_params=pltpu.CompilerParams(use_tc_tiling_on_sc=False),
    )(functools.partial(_agg_body, np_, ch, dw))


def _elu(v):
    return jnp.where(v > 0, v, jnp.exp(v) - 1.0)


def _mm1_body(x_ref, w_ref, d0_ref, d1_ref, g_ref):
    dinv = lax.rsqrt(d0_ref[...] + d1_ref[...] + 1.0)
    g_ref[...] = dinv * jnp.dot(x_ref[...], w_ref[...],
                                preferred_element_type=jnp.float32)


def _mm2_body(a0_ref, a1_ref, g1_ref, w_ref, d0_ref, d1_ref, g2_ref):
    dinv = lax.rsqrt(d0_ref[...] + d1_ref[...] + 1.0)
    h1 = _elu(dinv * (a0_ref[...] + a1_ref[...] + g1_ref[...]))
    g2_ref[...] = dinv * jnp.dot(h1, w_ref[...],
                                 preferred_element_type=jnp.float32)


def _final_body(nblk, a0_ref, a1_ref, g2_ref, d0_ref, d1_ref,
                bv_ref, lo_ref, hi_ref, l1w_ref, l1b_ref, l2w_ref, l2b_ref,
                out_ref, h2_ref, pool_ref):
    B = 512

    def h2_blk(b, carry):
        rs = pl.ds(b * B, B)
        dinv = lax.rsqrt(d0_ref[rs, :] + d1_ref[rs, :] + 1.0)
        h2_ref[rs, :] = _elu(dinv * (a0_ref[rs, :] + a1_ref[rs, :]
                                     + g2_ref[rs, :]))
        return carry

    lax.fori_loop(0, nblk, h2_blk, 0)

    pool_ref[...] = jnp.full((64, 128), -jnp.inf, dtype=jnp.float32)

    def seg_blk(b, carry):
        rs = pl.ds(b * B, B)
        bb = bv_ref[rs, :]
        hb = h2_ref[rs, :]
        glo = lo_ref[0, b]
        ghi = jnp.minimum(hi_ref[0, b], 63)

        def one_g(g, carry2):
            m = jnp.max(jnp.where(bb == g, hb, -jnp.inf), axis=0,
                        keepdims=True)
            pool_ref[pl.ds(g, 1), :] = jnp.maximum(pool_ref[pl.ds(g, 1), :], m)
            return carry2

        lax.fori_loop(glo, ghi + 1, one_g, 0)
        return carry

    lax.fori_loop(0, nblk, seg_blk, 0)

    p = _elu(pool_ref[...])
    h = _elu(jnp.dot(p, l1w_ref[...], preferred_element_type=jnp.float32)
             + l1b_ref[...])
    o = jnp.dot(h, l2w_ref[...], preferred_element_type=jnp.float32) \
        + l2b_ref[...]
    out_ref[...] = 1.0 / (1.0 + jnp.exp(-o))


def kernel(x, edge_index, batch, W1, W2, L1_w, L1_b, L2_w, L2_b):
    n = x.shape[0]
    e = edge_index.shape[1]
    d = x.shape[1]

    BLK = 512
    np_ = -(-n // 2048) * 2048
    nblk = np_ // BLK
    ch = -(-e // (NW * K))
    ep = NW * K * ch
    sink = n

    src = jnp.concatenate(
        [edge_index[0], jnp.zeros((ep - e,), jnp.int32)]).reshape(NW, ch, K)
    dst = jnp.concatenate(
        [edge_index[1], jnp.full((ep - e,), sink, jnp.int32)]).reshape(NW, ch, K)

    xp = jnp.pad(x, ((0, np_ - n), (0, 0)))
    batch_p = jnp.pad(batch, (0, np_ - n), constant_values=64)[:, None]
    blk_lo = batch_p[0::BLK, 0][None, :]
    blk_hi = batch_p[BLK - 1::BLK, 0][None, :]

    ones16 = jnp.ones((K, 16), jnp.float32)
    zeros16 = jnp.zeros((np_ // NS, 16), jnp.float32)
    dw = d // 2
    zeros_dw = jnp.zeros((K, dw), jnp.float32)

    deg16 = _make_deg_kernel(np_, ch)(dst, ones16, zeros16)
    d0 = deg16[0, :, :1]
    d1 = deg16[1, :, :1]

    row_spec = pl.BlockSpec((BLK, d), lambda i: (i, 0))
    w_spec = pl.BlockSpec((d, d), lambda i: (0, 0))
    deg_spec = pl.BlockSpec((BLK, 1), lambda i: (i, 0))
    g1 = pl.pallas_call(
        _mm1_body,
        grid=(nblk,),
        in_specs=[row_spec, w_spec, deg_spec, deg_spec],
        out_specs=row_spec,
        out_shape=jax.ShapeDtypeStruct((np_, d), jnp.float32),
    )(xp, W1, d0, d1)

    agg_k = _make_agg_kernel(np_, ch, dw)

    def agg(g):
        L = agg_k(jnp.asarray(g[:, :dw]), src, dst, zeros_dw)
        R = agg_k(jnp.asarray(g[:, dw:]), src, dst, zeros_dw)
        return jnp.concatenate([L, R], axis=2)

    agg1 = agg(g1)

    g2 = pl.pallas_call(
        _mm2_body,
        grid=(nblk,),
        in_specs=[row_spec, row_spec, row_spec, w_spec, deg_spec, deg_spec],
        out_specs=row_spec,
        out_shape=jax.ShapeDtypeStruct((np_, d), jnp.float32),
    )(agg1[0], agg1[1], g1, W2, d0, d1)

    agg2 = agg(g2)

    full = lambda shp: pl.BlockSpec(shp, lambda: (0,) * len(shp))
    out = pl.pallas_call(
        functools.partial(_final_body, nblk),
        in_specs=[full((np_, d)), full((np_, d)), full((np_, d)),
                  full((np_, 1)), full((np_, 1)),
                  full((np_, 1)),
                  pl.BlockSpec(memory_space=pltpu.SMEM),
                  pl.BlockSpec(memory_space=pltpu.SMEM),
                  full((d, 64)), full((1, 64)), full((64, 1)), full((1, 1))],
        out_specs=full((64, 1)),
        out_shape=jax.ShapeDtypeStruct((64, 1), jnp.float32),
        scratch_shapes=[pltpu.VMEM((np_, d), jnp.float32),
                        pltpu.VMEM((64, 128), jnp.float32)],
    )(agg2[0], agg2[1], g2, d0, d1, batch_p, blk_lo, blk_hi,
      L1_w, L1_b[None, :], L2_w, L2_b[None, :])

    return out


# speedup vs baseline: 13.0510x; 1.2995x over previous
import functools

import jax
import jax.numpy as jnp
from jax import lax
from jax.experimental import pallas as pl
from jax.experimental.pallas import tpu as pltpu
from jax.experimental.pallas import tpu_sc as plsc

NC = 2
NS = 16
NW = NC * NS
K = 128

_MESH = plsc.VectorSubcoreMesh(core_axis_name="c", subcore_axis_name="s")


def _deg_body(np_, ch, dst_hbm, ones_hbm, zeros_hbm, out_hbm,
              didx, ones_v, acc_sh, sem):
    c = lax.axis_index("c")
    s = lax.axis_index("s")
    w = c * NS + s
    rows = np_ // NS

    pltpu.sync_copy(ones_hbm, ones_v)
    pltpu.sync_copy(zeros_hbm, acc_sh.at[pl.ds(s * rows, rows)])
    plsc.subcore_barrier()

    def chunk(j, carry):
        pltpu.sync_copy(dst_hbm.at[w, j], didx)
        pltpu.sync_copy(ones_v, acc_sh.at[didx], add=True)
        return carry

    lax.fori_loop(0, ch, chunk, 0)
    plsc.subcore_barrier()
    pltpu.sync_copy(acc_sh.at[pl.ds(s * rows, rows)],
                    out_hbm.at[c, pl.ds(s * rows, rows)])


def _make_deg_kernel(np_, ch):
    return functools.partial(
        pl.kernel,
        out_type=jax.ShapeDtypeStruct((NC, np_, 16), jnp.float32),
        mesh=_MESH,
        scratch_types=[
            pltpu.VMEM((K,), jnp.int32),
            pltpu.VMEM((K, 16), jnp.float32),
            pltpu.VMEM_SHARED((np_, 16), jnp.float32),
            pltpu.SemaphoreType.DMA,
        ],
        compiler_params=pltpu.CompilerParams(use_tc_tiling_on_sc=False),
    )(functools.partial(_deg_body, np_, ch))


def _agg_body(np_, ch, dw, g_hbm, src_hbm, dst_hbm, zeros_hbm, out_hbm,
              sidx0, sidx1, didx0, msg0, msg1, acc_sh, gsem0, gsem1):
    c = lax.axis_index("c")
    s = lax.axis_index("s")
    w = c * NS + s
    rows = np_ // NS

    for r in range(rows // K):
        pltpu.sync_copy(zeros_hbm,
                        acc_sh.at[pl.ds(s * rows + r * K, K)])
    plsc.subcore_barrier()

    pltpu.sync_copy(src_hbm.at[w, 0], sidx0)
    pltpu.async_copy(g_hbm.at[sidx0], msg0, gsem0)

    def chunk(j, carry):
        def do(sidx_n, msg_n, gsem_n, sidx_c, msg_c, gsem_c):
            @pl.when(j + 1 < ch)
            def _():
                pltpu.sync_copy(src_hbm.at[w, j + 1], sidx_n)
                pltpu.async_copy(g_hbm.at[sidx_n], msg_n, gsem_n)
            pltpu.make_async_copy(g_hbm.at[sidx_c], msg_c, gsem_c).wait()
            pltpu.sync_copy(dst_hbm.at[w, j], didx0)
            pltpu.sync_copy(msg_c, acc_sh.at[didx0], add=True)

        @pl.when(j % 2 == 0)
        def _():
            do(sidx1, msg1, gsem1, sidx0, msg0, gsem0)

        @pl.when(j % 2 == 1)
        def _():
            do(sidx0, msg0, gsem0, sidx1, msg1, gsem1)

        return carry

    lax.fori_loop(0, ch, chunk, 0)
    plsc.subcore_barrier()
    pltpu.sync_copy(acc_sh.at[pl.ds(s * rows, rows)],
                    out_hbm.at[c, pl.ds(s * rows, rows)])


def _make_agg_kernel(np_, ch, dw):
    return functools.partial(
        pl.kernel,
        out_type=jax.ShapeDtypeStruct((NC, np_, dw), jnp.float32),
        mesh=_MESH,
        scratch_types=[
            pltpu.VMEM((K,), jnp.int32),
            pltpu.VMEM((K,), jnp.int32),
            pltpu.VMEM((K,), jnp.int32),
            pltpu.VMEM((K, dw), jnp.float32),
            pltpu.VMEM((K, dw), jnp.float32),
            pltpu.VMEM_SHARED((np_, dw), jnp.float32),
            pltpu.SemaphoreType.DMA,
            pltpu.SemaphoreType.DMA,
        ],
        compiler_params=pltpu.CompilerParams(use_tc_tiling_on_sc=False),
    )(functools.partial(_agg_body, np_, ch, dw))


def _elu(v):
    return jnp.where(v > 0, v, jnp.exp(v) - 1.0)


def _mm1_body(x_ref, w_ref, d0_ref, d1_ref, g_ref):
    dinv = lax.rsqrt(d0_ref[...] + d1_ref[...] + 1.0)
    g_ref[...] = dinv * jnp.dot(x_ref[...], w_ref[...],
                                preferred_element_type=jnp.float32)


def _mm2_body(a0_ref, a1_ref, g1_ref, w_ref, d0_ref, d1_ref, g2_ref):
    dinv = lax.rsqrt(d0_ref[...] + d1_ref[...] + 1.0)
    h1 = _elu(dinv * (a0_ref[...] + a1_ref[...] + g1_ref[...]))
    g2_ref[...] = dinv * jnp.dot(h1, w_ref[...],
                                 preferred_element_type=jnp.float32)


def _final_body(nblk, a0_ref, a1_ref, g2_ref, d0_ref, d1_ref,
                bv_ref, lo_ref, hi_ref, l1w_ref, l1b_ref, l2w_ref, l2b_ref,
                out_ref, h2_ref, pool_ref):
    B = 512

    def h2_blk(b, carry):
        rs = pl.ds(b * B, B)
        dinv = lax.rsqrt(d0_ref[rs, :] + d1_ref[rs, :] + 1.0)
        h2_ref[rs, :] = _elu(dinv * (a0_ref[rs, :] + a1_ref[rs, :]
                                     + g2_ref[rs, :]))
        return carry

    lax.fori_loop(0, nblk, h2_blk, 0)

    pool_ref[...] = jnp.full((64, 128), -jnp.inf, dtype=jnp.float32)

    def seg_blk(b, carry):
        rs = pl.ds(b * B, B)
        bb = bv_ref[rs, :]
        hb = h2_ref[rs, :]
        glo = lo_ref[0, b]
        ghi = jnp.minimum(hi_ref[0, b], 63)

        def one_g(g, carry2):
            m = jnp.max(jnp.where(bb == g, hb, -jnp.inf), axis=0,
                        keepdims=True)
            pool_ref[pl.ds(g, 1), :] = jnp.maximum(pool_ref[pl.ds(g, 1), :], m)
            return carry2

        lax.fori_loop(glo, ghi + 1, one_g, 0)
        return carry

    lax.fori_loop(0, nblk, seg_blk, 0)

    p = _elu(pool_ref[...])
    h = _elu(jnp.dot(p, l1w_ref[...], preferred_element_type=jnp.float32)
             + l1b_ref[...])
    o = jnp.dot(h, l2w_ref[...], preferred_element_type=jnp.float32) \
        + l2b_ref[...]
    out_ref[...] = 1.0 / (1.0 + jnp.exp(-o))


def kernel(x, edge_index, batch, W1, W2, L1_w, L1_b, L2_w, L2_b):
    n = x.shape[0]
    e = edge_index.shape[1]
    d = x.shape[1]

    BLK = 512
    np_ = -(-n // 2048) * 2048
    nblk = np_ // BLK
    ch = -(-e // (NW * K))
    ep = NW * K * ch
    sink = n

    src = jnp.concatenate(
        [edge_index[0], jnp.zeros((ep - e,), jnp.int32)]).reshape(NW, ch, K)
    dst = jnp.concatenate(
        [edge_index[1], jnp.full((ep - e,), sink, jnp.int32)]).reshape(NW, ch, K)

    xp = jnp.pad(x, ((0, np_ - n), (0, 0)))
    batch_p = jnp.pad(batch, (0, np_ - n), constant_values=64)[:, None]
    blk_lo = batch_p[0::BLK, 0][None, :]
    blk_hi = batch_p[BLK - 1::BLK, 0][None, :]

    ones16 = jnp.ones((K, 16), jnp.float32)
    zeros16 = jnp.zeros((np_ // NS, 16), jnp.float32)
    zeros_d = jnp.zeros((K, d), jnp.float32)

    deg16 = _make_deg_kernel(np_, ch)(dst, ones16, zeros16)
    d0 = deg16[0, :, :1]
    d1 = deg16[1, :, :1]

    row_spec = pl.BlockSpec((BLK, d), lambda i: (i, 0))
    w_spec = pl.BlockSpec((d, d), lambda i: (0, 0))
    deg_spec = pl.BlockSpec((BLK, 1), lambda i: (i, 0))
    g1 = pl.pallas_call(
        _mm1_body,
        grid=(nblk,),
        in_specs=[row_spec, w_spec, deg_spec, deg_spec],
        out_specs=row_spec,
        out_shape=jax.ShapeDtypeStruct((np_, d), jnp.float32),
    )(xp, W1, d0, d1)

    agg = _make_agg_kernel(np_, ch, d)
    agg1 = agg(g1, src, dst, zeros_d)

    g2 = pl.pallas_call(
        _mm2_body,
        grid=(nblk,),
        in_specs=[row_spec, row_spec, row_spec, w_spec, deg_spec, deg_spec],
        out_specs=row_spec,
        out_shape=jax.ShapeDtypeStruct((np_, d), jnp.float32),
    )(agg1[0], agg1[1], g1, W2, d0, d1)

    agg2 = agg(g2, src, dst, zeros_d)

    full = lambda shp: pl.BlockSpec(shp, lambda: (0,) * len(shp))
    out = pl.pallas_call(
        functools.partial(_final_body, nblk),
        in_specs=[full((np_, d)), full((np_, d)), full((np_, d)),
                  full((np_, 1)), full((np_, 1)),
                  full((np_, 1)),
                  pl.BlockSpec(memory_space=pltpu.SMEM),
                  pl.BlockSpec(memory_space=pltpu.SMEM),
                  full((d, 64)), full((1, 64)), full((64, 1)), full((1, 1))],
        out_specs=full((64, 1)),
        out_shape=jax.ShapeDtypeStruct((64, 1), jnp.float32),
        scratch_shapes=[pltpu.VMEM((np_, d), jnp.float32),
                        pltpu.VMEM((64, 128), jnp.float32)],
    )(agg2[0], agg2[1], g2, d0, d1, batch_p, blk_lo, blk_hi,
      L1_w, L1_b[None, :], L2_w, L2_b[None, :])

    return out
